# baseline (speedup 1.0000x reference)
import functools

import jax
import jax.numpy as jnp
from jax import lax
from jax.experimental import pallas as pl
from jax.experimental.pallas import tpu as pltpu
from jax.experimental.pallas import tpu_sc as plsc

_D = 64
_L = 200
_B = 4096
_NW = 32
_ROWS = _B * _L
_RPW = _ROWS // _NW
_S = 4
_CH = _S * _L
_NCH = _RPW // _CH
_NV = _D // 16


def _add_pos(rows_v, pos_v):
    def lbody(l, carry):
        pvs = [pos_v[l, pl.ds(c * 16, 16)] for c in range(_NV)]
        for s in range(_S):
            r = s * _L + l
            for c in range(_NV):
                rows_v[r, pl.ds(c * 16, 16)] = (
                    rows_v[r, pl.ds(c * 16, 16)] + pvs[c]
                )
        return carry
    lax.fori_loop(0, _L, lbody, 0, unroll=2)


def _sc_body(x_hbm, tok_hbm, pos_hbm, out_hbm,
             pos_v, idx0, idx1, rows0, rows1,
             isem0, isem1, gsem0, gsem1, wsem0, wsem1):
    wid = lax.axis_index("s") * 2 + lax.axis_index("c")
    wbase = wid * _RPW

    bufs = ((idx0, rows0, isem0, gsem0, wsem0),
            (idx1, rows1, isem1, gsem1, wsem1))

    pltpu.sync_copy(pos_hbm, pos_v)

    def idx_start(g, idx_b, isem_b):
        pltpu.async_copy(x_hbm.at[pl.ds(wbase + g * _CH, _CH)], idx_b, isem_b)

    def idx_wait(g, idx_b, isem_b):
        pltpu.make_async_copy(
            x_hbm.at[pl.ds(wbase + g * _CH, _CH)], idx_b, isem_b).wait()

    def gather_start(idx_b, rows_b, gsem_b):
        pltpu.async_copy(tok_hbm.at[idx_b], rows_b, gsem_b)

    def gather_wait(idx_b, rows_b, gsem_b):
        pltpu.make_async_copy(tok_hbm.at[idx_b], rows_b, gsem_b).wait()

    def write_start(g, rows_b, wsem_b):
        pltpu.async_copy(
            rows_b, out_hbm.at[pl.ds(wbase + g * _CH, _CH), pl.ds(0, _D)],
            wsem_b)

    def write_wait(g, rows_b, wsem_b):
        pltpu.make_async_copy(
            rows_b, out_hbm.at[pl.ds(wbase + g * _CH, _CH), pl.ds(0, _D)],
            wsem_b).wait()

    idx_start(0, idx0, isem0)
    idx_wait(0, idx0, isem0)
    gather_start(idx0, rows0, gsem0)
    idx_start(1, idx1, isem1)

    def outer(i, carry):
        for b in range(2):
            g = i * 2 + b
            idx_b, rows_b, isem_b, gsem_b, wsem_b = bufs[b]
            idx_n, rows_n, isem_n, gsem_n, wsem_n = bufs[1 - b]

            @pl.when(g + 1 < _NCH)
            def _start_next_gather():
                @pl.when(g >= 1)
                def _drain():
                    write_wait(g - 1, rows_n, wsem_n)
                idx_wait(g + 1, idx_n, isem_n)
                gather_start(idx_n, rows_n, gsem_n)

            gather_wait(idx_b, rows_b, gsem_b)

            @pl.when(g + 2 < _NCH)
            def _start_next_idx():
                idx_start(g + 2, idx_b, isem_b)

            _add_pos(rows_b, pos_v)
            write_start(g, rows_b, wsem_b)
        return carry

    lax.fori_loop(0, _NCH // 2, outer, 0)

    write_wait(_NCH - 2, rows0, wsem0)
    write_wait(_NCH - 1, rows1, wsem1)


@jax.jit
def _sc_embed(x, token_table, pos_table):
    mesh = plsc.VectorSubcoreMesh(core_axis_name="c", subcore_axis_name="s")
    out = pl.kernel(
        _sc_body,
        mesh=mesh,
        out_type=jax.ShapeDtypeStruct((_ROWS, 128), jnp.float32),
        compiler_params=pltpu.CompilerParams(use_tc_tiling_on_sc=False),
        scratch_types=[
            pltpu.VMEM((_L, _D), jnp.float32),
            pltpu.VMEM((_CH,), jnp.int32),
            pltpu.VMEM((_CH,), jnp.int32),
            pltpu.VMEM((_CH, _D), jnp.float32),
            pltpu.VMEM((_CH, _D), jnp.float32),
            pltpu.SemaphoreType.DMA,
            pltpu.SemaphoreType.DMA,
            pltpu.SemaphoreType.DMA,
            pltpu.SemaphoreType.DMA,
            pltpu.SemaphoreType.DMA,
            pltpu.SemaphoreType.DMA,
        ],
    )(x.reshape(_ROWS), token_table, pos_table)
    return out[:, :_D].reshape(_B, _L, _D)


def kernel(x, token_table, pos_table):
    return _sc_embed(x, token_table, pos_table)
